# Initial kernel scaffold; baseline (speedup 1.0000x reference)
import jax
import jax.numpy as jnp
from jax.experimental import pallas as pl


def kernel(x, conv1_w, conv1_b, b0_c1_w, b0_c1_b, b0_c2_w, b0_c2_b, b0_c3_w, b0_c3_b, b0_down_w, b0_down_b, b1_c1_w, b1_c1_b, b1_c2_w, b1_c2_b, b1_c3_w, b1_c3_b, b1_down_w, b1_down_b, b2_c1_w, b2_c1_b, b2_c2_w, b2_c2_b, b2_c3_w, b2_c3_b, b2_down_w, b2_down_b, b3_c1_w, b3_c1_b, b3_c2_w, b3_c2_b, b3_c3_w, b3_c3_b, b3_down_w, b3_down_b, fc_w, fc_b):
    raise NotImplementedError("write your pallas kernel here")


# speedup vs baseline: 1.0035x; 1.0035x over previous
import functools

import jax
import jax.numpy as jnp
from jax.experimental import pallas as pl
from jax.experimental.pallas import tpu as pltpu


def _round_up(x, m):
    return (x + m - 1) // m * m


def _mm_kernel(a_ref, w_ref, b_ref, *rest, relu, has_res):
    if has_res:
        r_ref, o_ref = rest
    else:
        (o_ref,) = rest
    acc = jnp.dot(a_ref[...], w_ref[...], preferred_element_type=jnp.float32)
    acc = acc + b_ref[...]
    if has_res:
        acc = acc + r_ref[...].astype(jnp.float32)
    if relu:
        acc = jnp.maximum(acc, 0.0)
    o_ref[...] = acc.astype(o_ref.dtype)


def _matmul_bias(a, w, bias, *, relu, residual=None, out_dtype=jnp.bfloat16,
                 tm=512, tn=512):
    M, K = a.shape
    N = w.shape[1]
    tm = min(tm, _round_up(M, 8))
    Mp = _round_up(M, tm)
    Kp = _round_up(K, 16)
    tn = min(tn, N)
    Np = _round_up(N, tn)

    a_p = a.astype(jnp.bfloat16)
    if (Mp, Kp) != (M, K):
        a_p = jnp.pad(a_p, ((0, Mp - M), (0, Kp - K)))
    w_p = w.astype(jnp.bfloat16)
    if (Kp, Np) != (K, N):
        w_p = jnp.pad(w_p, ((0, Kp - K), (0, Np - N)))
    b_p = bias.astype(jnp.float32)
    if Np != N:
        b_p = jnp.pad(b_p, (0, Np - N))
    b_p = b_p.reshape(1, Np)

    inputs = [a_p, w_p, b_p]
    in_specs = [
        pl.BlockSpec((tm, Kp), lambda i, j: (i, 0)),
        pl.BlockSpec((Kp, tn), lambda i, j: (0, j)),
        pl.BlockSpec((1, tn), lambda i, j: (0, j)),
    ]
    has_res = residual is not None
    if has_res:
        r_p = residual.astype(jnp.bfloat16)
        if (Mp, Np) != (M, N):
            r_p = jnp.pad(r_p, ((0, Mp - M), (0, Np - N)))
        inputs.append(r_p)
        in_specs.append(pl.BlockSpec((tm, tn), lambda i, j: (i, j)))

    out = pl.pallas_call(
        functools.partial(_mm_kernel, relu=relu, has_res=has_res),
        out_shape=jax.ShapeDtypeStruct((Mp, Np), out_dtype),
        grid=(Mp // tm, Np // tn),
        in_specs=in_specs,
        out_specs=pl.BlockSpec((tm, tn), lambda i, j: (i, j)),
        compiler_params=pltpu.CompilerParams(
            dimension_semantics=("parallel", "parallel"),
            vmem_limit_bytes=100 * 1024 * 1024,
        ),
    )(*inputs)
    if (Mp, Np) != (M, N):
        out = out[:M, :N]
    return out


def _im2col(x, kh, kw, stride, padding):
    N, H, W, C = x.shape
    Ho = (H + 2 * padding - kh) // stride + 1
    Wo = (W + 2 * padding - kw) // stride + 1
    xp = jnp.pad(x, ((0, 0), (padding, padding), (padding, padding), (0, 0)))
    cols = []
    for i in range(kh):
        for j in range(kw):
            cols.append(xp[:, i:i + stride * Ho:stride, j:j + stride * Wo:stride, :])
    patches = jnp.concatenate(cols, axis=-1)
    return patches.reshape(N * Ho * Wo, kh * kw * C), (N, Ho, Wo)


def _conv(x, w_hwio, bias, *, stride, padding, relu, residual=None):
    kh, kw, cin, cout = w_hwio.shape
    if kh == 1 and kw == 1 and stride == 1 and padding == 0:
        N, H, W, _ = x.shape
        a = x.reshape(N * H * W, cin)
        shape_out = (N, H, W)
    elif kh == 1 and kw == 1 and padding == 0:
        a = x[:, ::stride, ::stride, :]
        N, Ho, Wo, _ = a.shape
        a = a.reshape(N * Ho * Wo, cin)
        shape_out = (N, Ho, Wo)
    else:
        a, shape_out = _im2col(x, kh, kw, stride, padding)
    N, Ho, Wo = shape_out
    w2d = w_hwio.reshape(kh * kw * cin, cout)
    res2d = None if residual is None else residual.reshape(N * Ho * Wo, cout)
    out = _matmul_bias(a, w2d, bias, relu=relu, residual=res2d)
    return out.reshape(N, Ho, Wo, cout)


def _maxpool_3x3_s2_p1(x):
    N, H, W, C = x.shape
    Ho = (H - 1) // 2 + 1
    Wo = (W - 1) // 2 + 1
    xp = jnp.pad(x, ((0, 0), (1, 1), (1, 1), (0, 0)), constant_values=-jnp.inf)
    out = None
    for i in range(3):
        for j in range(3):
            tap = xp[:, i:i + 2 * Ho:2, j:j + 2 * Wo:2, :]
            out = tap if out is None else jnp.maximum(out, tap)
    return out


def _pool_fc_kernel(x_ref, w_ref, b_ref, o_ref, *, hw):
    feats = jnp.sum(x_ref[...].astype(jnp.float32), axis=1) * (1.0 / hw)
    o_ref[...] = jnp.dot(feats, w_ref[...],
                         preferred_element_type=jnp.float32) + b_ref[...]


def _avgpool_fc(x, fc_w, fc_b):
    N, H, W, C = x.shape
    ncls = fc_w.shape[1]
    HWp = _round_up(H * W, 8)
    Np_cls = _round_up(ncls, 128)
    x3 = x.reshape(N, H * W, C)
    if HWp != H * W:
        x3 = jnp.pad(x3, ((0, 0), (0, HWp - H * W), (0, 0)))
    w_p = jnp.pad(fc_w.astype(jnp.float32), ((0, 0), (0, Np_cls - ncls)))
    b_p = jnp.pad(fc_b.astype(jnp.float32), (0, Np_cls - ncls)).reshape(1, Np_cls)
    out = pl.pallas_call(
        functools.partial(_pool_fc_kernel, hw=float(H * W)),
        out_shape=jax.ShapeDtypeStruct((N, Np_cls), jnp.float32),
        grid=(1,),
        in_specs=[
            pl.BlockSpec((N, HWp, C), lambda i: (0, 0, 0)),
            pl.BlockSpec((C, Np_cls), lambda i: (0, 0)),
            pl.BlockSpec((1, Np_cls), lambda i: (0, 0)),
        ],
        out_specs=pl.BlockSpec((N, Np_cls), lambda i: (0, 0)),
        compiler_params=pltpu.CompilerParams(
            vmem_limit_bytes=100 * 1024 * 1024),
    )(x3, w_p, b_p)
    return out[:, :ncls]


def _bottleneck(x, c1, c2, c3, down, stride):
    identity = _conv(x, down[0], down[1], stride=stride, padding=0, relu=False)
    out = _conv(x, c1[0], c1[1], stride=1, padding=0, relu=True)
    out = _conv(out, c2[0], c2[1], stride=stride, padding=1, relu=True)
    out = _conv(out, c3[0], c3[1], stride=1, padding=0, relu=True,
                residual=identity)
    return out


def kernel(x, conv1_w, conv1_b,
           b0_c1_w, b0_c1_b, b0_c2_w, b0_c2_b, b0_c3_w, b0_c3_b,
           b0_down_w, b0_down_b,
           b1_c1_w, b1_c1_b, b1_c2_w, b1_c2_b, b1_c3_w, b1_c3_b,
           b1_down_w, b1_down_b,
           b2_c1_w, b2_c1_b, b2_c2_w, b2_c2_b, b2_c3_w, b2_c3_b,
           b2_down_w, b2_down_b,
           b3_c1_w, b3_c1_b, b3_c2_w, b3_c2_b, b3_c3_w, b3_c3_b,
           b3_down_w, b3_down_b,
           fc_w, fc_b):
    h = jnp.transpose(x, (0, 2, 3, 1)).astype(jnp.bfloat16)
    h = _conv(h, conv1_w, conv1_b, stride=2, padding=3, relu=True)
    h = _maxpool_3x3_s2_p1(h)
    blocks = [
        ((b0_c1_w, b0_c1_b), (b0_c2_w, b0_c2_b), (b0_c3_w, b0_c3_b),
         (b0_down_w, b0_down_b), 1),
        ((b1_c1_w, b1_c1_b), (b1_c2_w, b1_c2_b), (b1_c3_w, b1_c3_b),
         (b1_down_w, b1_down_b), 2),
        ((b2_c1_w, b2_c1_b), (b2_c2_w, b2_c2_b), (b2_c3_w, b2_c3_b),
         (b2_down_w, b2_down_b), 2),
        ((b3_c1_w, b3_c1_b), (b3_c2_w, b3_c2_b), (b3_c3_w, b3_c3_b),
         (b3_down_w, b3_down_b), 2),
    ]
    for c1, c2, c3, down, s in blocks:
        h = _bottleneck(h, c1, c2, c3, down, s)
    return _avgpool_fc(h, fc_w, fc_b)


# speedup vs baseline: 4.7004x; 4.6839x over previous
import functools

import jax
import jax.numpy as jnp
from jax.experimental import pallas as pl
from jax.experimental.pallas import tpu as pltpu


_VMEM = 100 * 1024 * 1024


def _stem_kernel(x_ref, w_ref, b_ref, o_ref, s_ref):
    x = x_ref[0]
    accs = [None, None]
    for a in range(4):
        for b in range(4):
            wt = w_ref[(a * 4 + b) * 12:(a * 4 + b + 1) * 12, :]
            for pj in range(2):
                cc = b + pj
                p0, hh = cc // 2, cc % 2
                tap = x[a:a + 112, p0:p0 + 56, hh * 12:(hh + 1) * 12]
                d = jnp.dot(tap.reshape(6272, 12), wt,
                            preferred_element_type=jnp.float32)
                accs[pj] = d if accs[pj] is None else accs[pj] + d
    bias = b_ref[...]
    ye = jnp.maximum(accs[0] + bias, 0.0)
    yo = jnp.maximum(accs[1] + bias, 0.0)
    s_ref[...] = jnp.zeros_like(s_ref)
    ye5 = ye.reshape(56, 2, 56, 64)
    yo5 = yo.reshape(56, 2, 56, 64)
    s_ref[1:57, 0, 1:57, 0:64] = ye5[:, 0]
    s_ref[1:57, 1, 1:57, 0:64] = ye5[:, 1]
    s_ref[1:57, 0, 1:57, 64:128] = yo5[:, 0]
    s_ref[1:57, 1, 1:57, 64:128] = yo5[:, 1]
    m = None
    row_idx = ((0, 1), (1, 0), (1, 1))
    col_idx = ((0, 64), (1, 0), (1, 64))
    for a0, pa in row_idx:
        for p0, l0 in col_idx:
            t = s_ref[a0:a0 + 56, pa, p0:p0 + 56, l0:l0 + 64]
            m = t if m is None else jnp.maximum(m, t)
    o_ref[...] = m[None].astype(jnp.bfloat16)


def _stem(x_nchw, conv1_w, conv1_b):
    N = x_nchw.shape[0]
    xt = jnp.transpose(x_nchw, (0, 2, 3, 1)).astype(jnp.bfloat16)
    xp = jnp.pad(xt, ((0, 0), (3, 3), (3, 3), (0, 0)))
    xs = xp.reshape(N, 115, 2, 115, 2, 3).transpose(0, 1, 3, 2, 4, 5)
    xs = xs.reshape(N, 115, 115, 12)
    xs = jnp.pad(xs, ((0, 0), (0, 0), (0, 1), (0, 0))).reshape(N, 115, 58, 24)
    w = jnp.pad(conv1_w.astype(jnp.float32), ((0, 1), (0, 1), (0, 0), (0, 0)))
    w4 = w.reshape(4, 2, 4, 2, 3, 64).transpose(0, 2, 1, 3, 4, 5)
    w4 = w4.reshape(16 * 12, 64).astype(jnp.bfloat16)
    b2 = conv1_b.astype(jnp.float32).reshape(1, 64)
    return pl.pallas_call(
        _stem_kernel,
        out_shape=jax.ShapeDtypeStruct((N, 56, 56, 64), jnp.bfloat16),
        grid=(N,),
        in_specs=[
            pl.BlockSpec((1, 115, 58, 24), lambda n: (n, 0, 0, 0)),
            pl.BlockSpec((192, 64), lambda n: (0, 0)),
            pl.BlockSpec((1, 64), lambda n: (0, 0)),
        ],
        out_specs=pl.BlockSpec((1, 56, 56, 64), lambda n: (n, 0, 0, 0)),
        scratch_shapes=[pltpu.VMEM((58, 2, 58, 128), jnp.float32)],
        compiler_params=pltpu.CompilerParams(
            dimension_semantics=("parallel",), vmem_limit_bytes=_VMEM),
    )(xs, w4, b2)


def _b0_kernel(x_ref, w1_ref, b1_ref, w2_ref, b2_ref, w3_ref, b3_ref,
               wd_ref, bd_ref, wn_ref, bn_ref, o_ref, yn_ref, y1p_ref):
    H, C, P, CO = 56, 64, 64, 256
    M = H * H
    x = x_ref[...].reshape(M, C)
    y1 = jnp.dot(x, w1_ref[...], preferred_element_type=jnp.float32)
    y1 = jnp.maximum(y1 + b1_ref[...], 0.0).astype(jnp.bfloat16)
    y1p_ref[...] = jnp.zeros_like(y1p_ref)
    y1p_ref[0, 1:H + 1, 1:H + 1, :] = y1.reshape(H, H, P)
    acc = None
    for dy in range(3):
        for dx in range(3):
            t = y1p_ref[0, dy:dy + H, dx:dx + H, :]
            wt = w2_ref[(dy * 3 + dx) * P:(dy * 3 + dx + 1) * P, :]
            d = jnp.dot(t.reshape(M, P), wt,
                        preferred_element_type=jnp.float32)
            acc = d if acc is None else acc + d
    y2 = jnp.maximum(acc + b2_ref[...], 0.0).astype(jnp.bfloat16)
    ident = jnp.dot(x, wd_ref[...],
                    preferred_element_type=jnp.float32) + bd_ref[...]
    out = jnp.dot(y2, w3_ref[...],
                  preferred_element_type=jnp.float32) + b3_ref[...]
    obf = jnp.maximum(out + ident, 0.0).astype(jnp.bfloat16)
    o_ref[...] = obf.reshape(1, H, H, CO)
    y1n = jnp.dot(obf, wn_ref[...], preferred_element_type=jnp.float32)
    y1n = jnp.maximum(y1n + bn_ref[...], 0.0).astype(jnp.bfloat16)
    yn_ref[...] = y1n.reshape(1, H, H, 128)


def _block0(x, c1, c2, c3, down, nxt):
    N = x.shape[0]
    const = lambda n: (0, 0)
    img = lambda n: (n, 0, 0, 0)
    return pl.pallas_call(
        _b0_kernel,
        out_shape=(
            jax.ShapeDtypeStruct((N, 56, 56, 256), jnp.bfloat16),
            jax.ShapeDtypeStruct((N, 56, 56, 128), jnp.bfloat16),
        ),
        grid=(N,),
        in_specs=[
            pl.BlockSpec((1, 56, 56, 64), img),
            pl.BlockSpec((64, 64), const),
            pl.BlockSpec((1, 64), const),
            pl.BlockSpec((576, 64), const),
            pl.BlockSpec((1, 64), const),
            pl.BlockSpec((64, 256), const),
            pl.BlockSpec((1, 256), const),
            pl.BlockSpec((64, 256), const),
            pl.BlockSpec((1, 256), const),
            pl.BlockSpec((256, 128), const),
            pl.BlockSpec((1, 128), const),
        ],
        out_specs=(
            pl.BlockSpec((1, 56, 56, 256), img),
            pl.BlockSpec((1, 56, 56, 128), img),
        ),
        scratch_shapes=[pltpu.VMEM((1, 58, 58, 64), jnp.bfloat16)],
        compiler_params=pltpu.CompilerParams(
            dimension_semantics=("parallel",), vmem_limit_bytes=_VMEM),
    )(x, c1[0].reshape(64, 64), c1[1].astype(jnp.float32).reshape(1, 64),
      c2[0].reshape(576, 64), c2[1].astype(jnp.float32).reshape(1, 64),
      c3[0].reshape(64, 256), c3[1].astype(jnp.float32).reshape(1, 256),
      down[0].reshape(64, 256), down[1].astype(jnp.float32).reshape(1, 256),
      nxt[0].reshape(256, 128), nxt[1].astype(jnp.float32).reshape(1, 128))


def _bs2_kernel(p00_ref, p01_ref, p10_ref, p11_ref, xd_ref,
                w2_ref, b2_ref, w3_ref, b3_ref, wd_ref, bd_ref, *rest,
                G, HO, WT, C, P, CO, PN):
    if PN:
        wn_ref, bn_ref, o_ref, yn_ref = rest
    else:
        (o_ref,) = rest
    M = G * HO * WT
    phases = (p00_ref, p01_ref, p10_ref, p11_ref)
    acc = None
    for dy in range(3):
        for dx in range(3):
            ph = phases[(dy % 2) * 2 + (dx % 2)]
            i0, j0 = dy // 2, dx // 2
            t = ph[:, i0:i0 + HO, j0:j0 + WT, :]
            wt = w2_ref[(dy * 3 + dx) * P:(dy * 3 + dx + 1) * P, :]
            d = jnp.dot(t.reshape(M, P), wt,
                        preferred_element_type=jnp.float32)
            acc = d if acc is None else acc + d
    y2 = jnp.maximum(acc + b2_ref[...], 0.0).astype(jnp.bfloat16)
    ident = jnp.dot(xd_ref[...].reshape(M, C), wd_ref[...],
                    preferred_element_type=jnp.float32) + bd_ref[...]
    out = jnp.dot(y2, w3_ref[...],
                  preferred_element_type=jnp.float32) + b3_ref[...]
    obf = jnp.maximum(out + ident, 0.0).astype(jnp.bfloat16)
    o_ref[...] = obf.reshape(G, HO, WT, CO)[:, :, :HO, :]
    if PN:
        y1n = jnp.dot(obf, wn_ref[...], preferred_element_type=jnp.float32)
        y1n = jnp.maximum(y1n + bn_ref[...], 0.0).astype(jnp.bfloat16)
        yn_ref[...] = y1n.reshape(G, HO, WT, PN)[:, :, :HO, :]


def _phase_split(y1, WT):
    N, H, _, P = y1.shape
    hp = (H + 2) // 2
    yp = jnp.pad(y1, ((0, 0), (1, 1), (1, 1), (0, 0)))
    out = []
    for py in range(2):
        for px in range(2):
            q = yp[:, py::2, px::2, :]
            out.append(jnp.pad(q, ((0, 0), (0, 0), (0, WT + 1 - hp), (0, 0))))
    return out


def _block_s2(x, y1, c2, c3, down, nxt, *, G, WT):
    N, H, _, C = x.shape
    P = y1.shape[3]
    CO = c3[0].shape[3]
    PN = nxt[0].shape[3] if nxt is not None else 0
    HO = H // 2
    phases = _phase_split(y1, WT)
    xd = x[:, ::2, ::2, :]
    xd = jnp.pad(xd, ((0, 0), (0, 0), (0, WT - HO), (0, 0)))
    hp = (H + 2) // 2
    const = lambda n: (0, 0)
    img = lambda n: (n, 0, 0, 0)
    inputs = phases + [
        xd,
        c2[0].reshape(9 * P, P), c2[1].astype(jnp.float32).reshape(1, P),
        c3[0].reshape(P, CO), c3[1].astype(jnp.float32).reshape(1, CO),
        down[0].reshape(C, CO), down[1].astype(jnp.float32).reshape(1, CO),
    ]
    in_specs = [pl.BlockSpec((G, hp, WT + 1, P), img) for _ in range(4)] + [
        pl.BlockSpec((G, HO, WT, C), img),
        pl.BlockSpec((9 * P, P), const), pl.BlockSpec((1, P), const),
        pl.BlockSpec((P, CO), const), pl.BlockSpec((1, CO), const),
        pl.BlockSpec((C, CO), const), pl.BlockSpec((1, CO), const),
    ]
    out_shape = [jax.ShapeDtypeStruct((N, HO, HO, CO), jnp.bfloat16)]
    out_specs = [pl.BlockSpec((G, HO, HO, CO), img)]
    if PN:
        inputs += [nxt[0].reshape(CO, PN),
                   nxt[1].astype(jnp.float32).reshape(1, PN)]
        in_specs += [pl.BlockSpec((CO, PN), const),
                     pl.BlockSpec((1, PN), const)]
        out_shape.append(jax.ShapeDtypeStruct((N, HO, HO, PN), jnp.bfloat16))
        out_specs.append(pl.BlockSpec((G, HO, HO, PN), img))
    res = pl.pallas_call(
        functools.partial(_bs2_kernel, G=G, HO=HO, WT=WT, C=C, P=P, CO=CO,
                          PN=PN),
        out_shape=tuple(out_shape),
        grid=(N // G,),
        in_specs=in_specs,
        out_specs=tuple(out_specs),
        compiler_params=pltpu.CompilerParams(
            dimension_semantics=("parallel",), vmem_limit_bytes=_VMEM),
    )(*inputs)
    return res if PN else (res[0], None)


def _pool_fc_kernel(x_ref, w_ref, b_ref, o_ref, *, hw):
    feats = jnp.sum(x_ref[...].astype(jnp.float32), axis=(1, 2)) * (1.0 / hw)
    o_ref[...] = jnp.dot(feats, w_ref[...],
                         preferred_element_type=jnp.float32) + b_ref[...]


def _avgpool_fc(x, fc_w, fc_b):
    N, H, W, C = x.shape
    ncls = fc_w.shape[1]
    Np = (ncls + 127) // 128 * 128
    w_p = jnp.pad(fc_w.astype(jnp.float32), ((0, 0), (0, Np - ncls)))
    b_p = jnp.pad(fc_b.astype(jnp.float32), (0, Np - ncls)).reshape(1, Np)
    out = pl.pallas_call(
        functools.partial(_pool_fc_kernel, hw=float(H * W)),
        out_shape=jax.ShapeDtypeStruct((N, Np), jnp.float32),
        grid=(1,),
        in_specs=[
            pl.BlockSpec((N, H, W, C), lambda i: (0, 0, 0, 0)),
            pl.BlockSpec((C, Np), lambda i: (0, 0)),
            pl.BlockSpec((1, Np), lambda i: (0, 0)),
        ],
        out_specs=pl.BlockSpec((N, Np), lambda i: (0, 0)),
        compiler_params=pltpu.CompilerParams(vmem_limit_bytes=_VMEM),
    )(x, w_p, b_p)
    return out[:, :ncls]


def kernel(x, conv1_w, conv1_b,
           b0_c1_w, b0_c1_b, b0_c2_w, b0_c2_b, b0_c3_w, b0_c3_b,
           b0_down_w, b0_down_b,
           b1_c1_w, b1_c1_b, b1_c2_w, b1_c2_b, b1_c3_w, b1_c3_b,
           b1_down_w, b1_down_b,
           b2_c1_w, b2_c1_b, b2_c2_w, b2_c2_b, b2_c3_w, b2_c3_b,
           b2_down_w, b2_down_b,
           b3_c1_w, b3_c1_b, b3_c2_w, b3_c2_b, b3_c3_w, b3_c3_b,
           b3_down_w, b3_down_b,
           fc_w, fc_b):
    h = _stem(x, conv1_w, conv1_b)
    h, y1 = _block0(h, (b0_c1_w, b0_c1_b), (b0_c2_w, b0_c2_b),
                    (b0_c3_w, b0_c3_b), (b0_down_w, b0_down_b),
                    (b1_c1_w, b1_c1_b))
    h, y1 = _block_s2(h, y1, (b1_c2_w, b1_c2_b), (b1_c3_w, b1_c3_b),
                      (b1_down_w, b1_down_b), (b2_c1_w, b2_c1_b),
                      G=1, WT=32)
    h, y1 = _block_s2(h, y1, (b2_c2_w, b2_c2_b), (b2_c3_w, b2_c3_b),
                      (b2_down_w, b2_down_b), (b3_c1_w, b3_c1_b),
                      G=2, WT=16)
    h, _ = _block_s2(h, y1, (b3_c2_w, b3_c2_b), (b3_c3_w, b3_c3_b),
                     (b3_down_w, b3_down_b), None,
                     G=8, WT=8)
    return _avgpool_fc(h, fc_w, fc_b)


# speedup vs baseline: 20.2778x; 4.3140x over previous
import functools

import jax
import jax.numpy as jnp
from jax.experimental import pallas as pl
from jax.experimental.pallas import tpu as pltpu


_VMEM = 100 * 1024 * 1024


def _stem_kernel(x_ref, w_ref, b_ref, o_ref, s_ref):
    x = x_ref[0]
    accs = [None, None]
    for a in range(4):
        for b in range(4):
            wt = w_ref[(a * 4 + b) * 12:(a * 4 + b + 1) * 12, :]
            for pj in range(2):
                cc = b + pj
                p0, hh = cc // 2, cc % 2
                tap = x[a:a + 112, p0:p0 + 56, hh * 12:(hh + 1) * 12]
                d = jnp.dot(tap.reshape(6272, 12), wt,
                            preferred_element_type=jnp.float32)
                accs[pj] = d if accs[pj] is None else accs[pj] + d
    bias = b_ref[...]
    ye = jnp.maximum(accs[0] + bias, 0.0)
    yo = jnp.maximum(accs[1] + bias, 0.0)
    s_ref[...] = jnp.zeros_like(s_ref)
    ye5 = ye.reshape(56, 2, 56, 64)
    yo5 = yo.reshape(56, 2, 56, 64)
    s_ref[1:57, 0, 1:57, 0:64] = ye5[:, 0]
    s_ref[1:57, 1, 1:57, 0:64] = ye5[:, 1]
    s_ref[1:57, 0, 1:57, 64:128] = yo5[:, 0]
    s_ref[1:57, 1, 1:57, 64:128] = yo5[:, 1]
    m = None
    row_idx = ((0, 1), (1, 0), (1, 1))
    col_idx = ((0, 64), (1, 0), (1, 64))
    for a0, pa in row_idx:
        for p0, l0 in col_idx:
            t = s_ref[a0:a0 + 56, pa, p0:p0 + 56, l0:l0 + 64]
            m = t if m is None else jnp.maximum(m, t)
    o_ref[...] = m[None].astype(jnp.bfloat16)


def _stem(x_nchw, conv1_w, conv1_b):
    N = x_nchw.shape[0]
    xt = jnp.transpose(x_nchw, (0, 2, 3, 1)).astype(jnp.bfloat16)
    xp = jnp.pad(xt, ((0, 0), (3, 3), (3, 3), (0, 0)))
    xs = xp.reshape(N, 115, 2, 115, 2, 3).transpose(0, 1, 3, 2, 4, 5)
    xs = xs.reshape(N, 115, 115, 12)
    xs = jnp.pad(xs, ((0, 0), (0, 0), (0, 1), (0, 0))).reshape(N, 115, 58, 24)
    w = jnp.pad(conv1_w.astype(jnp.float32), ((0, 1), (0, 1), (0, 0), (0, 0)))
    w4 = w.reshape(4, 2, 4, 2, 3, 64).transpose(0, 2, 1, 3, 4, 5)
    w4 = w4.reshape(16 * 12, 64).astype(jnp.bfloat16)
    b2 = conv1_b.astype(jnp.float32).reshape(1, 64)
    return pl.pallas_call(
        _stem_kernel,
        out_shape=jax.ShapeDtypeStruct((N, 56, 56, 64), jnp.bfloat16),
        grid=(N,),
        in_specs=[
            pl.BlockSpec((1, 115, 58, 24), lambda n: (n, 0, 0, 0)),
            pl.BlockSpec((192, 64), lambda n: (0, 0)),
            pl.BlockSpec((1, 64), lambda n: (0, 0)),
        ],
        out_specs=pl.BlockSpec((1, 56, 56, 64), lambda n: (n, 0, 0, 0)),
        scratch_shapes=[pltpu.VMEM((58, 2, 58, 128), jnp.float32)],
        compiler_params=pltpu.CompilerParams(
            dimension_semantics=("parallel",), vmem_limit_bytes=_VMEM),
    )(xs, w4, b2)


def _emit_phases(y, hp, WTp1):
    G, Hn, _, Pn = y.shape
    h2 = Hn // 2
    re = y.reshape(G, h2, 2, Hn, Pn)
    evg = re[:, :, 0].reshape(G, h2, h2, 2 * Pn)
    odg = re[:, :, 1].reshape(G, h2, h2, 2 * Pn)
    rpad = hp - h2
    cpad = WTp1 - h2
    p11 = jnp.pad(evg[..., :Pn], ((0, 0), (0, rpad), (0, cpad), (0, 0)))
    p10 = jnp.pad(evg[..., Pn:], ((0, 0), (0, rpad), (1, cpad - 1), (0, 0)))
    p01 = jnp.pad(odg[..., :Pn], ((0, 0), (rpad, 0), (0, cpad), (0, 0)))
    p00 = jnp.pad(odg[..., Pn:], ((0, 0), (rpad, 0), (1, cpad - 1), (0, 0)))
    return p00, p01, p10, p11


def _emit_xd(o, WT):
    G, Hn, _, CO = o.shape
    h2 = Hn // 2
    ev = o.reshape(G, h2, 2, Hn, CO)[:, :, 0].reshape(G, h2, h2, 2 * CO)
    return jnp.pad(ev[..., :CO], ((0, 0), (0, 0), (0, WT - h2), (0, 0)))


def _b0_kernel(x_ref, w1_ref, b1_ref, w2_ref, b2_ref, w3_ref, b3_ref,
               wd_ref, bd_ref, wn_ref, bn_ref,
               p00_ref, p01_ref, p10_ref, p11_ref, xd_ref, y1p_ref):
    H, C, P, CO = 56, 64, 64, 256
    M = H * H
    x = x_ref[...].reshape(M, C)
    y1 = jnp.dot(x, w1_ref[...], preferred_element_type=jnp.float32)
    y1 = jnp.maximum(y1 + b1_ref[...], 0.0).astype(jnp.bfloat16)
    y1p_ref[...] = jnp.zeros_like(y1p_ref)
    y1p_ref[0, 1:H + 1, 1:H + 1, :] = y1.reshape(H, H, P)
    acc = None
    for dy in range(3):
        for dx in range(3):
            t = y1p_ref[0, dy:dy + H, dx:dx + H, :]
            wt = w2_ref[(dy * 3 + dx) * P:(dy * 3 + dx + 1) * P, :]
            d = jnp.dot(t.reshape(M, P), wt,
                        preferred_element_type=jnp.float32)
            acc = d if acc is None else acc + d
    y2 = jnp.maximum(acc + b2_ref[...], 0.0).astype(jnp.bfloat16)
    ident = jnp.dot(x, wd_ref[...],
                    preferred_element_type=jnp.float32) + bd_ref[...]
    out = jnp.dot(y2, w3_ref[...],
                  preferred_element_type=jnp.float32) + b3_ref[...]
    obf = jnp.maximum(out + ident, 0.0).astype(jnp.bfloat16)
    y1n = jnp.dot(obf, wn_ref[...], preferred_element_type=jnp.float32)
    y1n = jnp.maximum(y1n + bn_ref[...], 0.0).astype(jnp.bfloat16)
    p00, p01, p10, p11 = _emit_phases(y1n.reshape(1, H, H, 128), 29, 33)
    p00_ref[...], p01_ref[...] = p00, p01
    p10_ref[...], p11_ref[...] = p10, p11
    xd_ref[...] = _emit_xd(obf.reshape(1, H, H, CO), 32)


def _block0(x, c1, c2, c3, down, nxt):
    N = x.shape[0]
    const = lambda n: (0, 0)
    img = lambda n: (n, 0, 0, 0)
    return pl.pallas_call(
        _b0_kernel,
        out_shape=(
            jax.ShapeDtypeStruct((N, 29, 33, 128), jnp.bfloat16),
            jax.ShapeDtypeStruct((N, 29, 33, 128), jnp.bfloat16),
            jax.ShapeDtypeStruct((N, 29, 33, 128), jnp.bfloat16),
            jax.ShapeDtypeStruct((N, 29, 33, 128), jnp.bfloat16),
            jax.ShapeDtypeStruct((N, 28, 32, 256), jnp.bfloat16),
        ),
        grid=(N,),
        in_specs=[
            pl.BlockSpec((1, 56, 56, 64), img),
            pl.BlockSpec((64, 64), const),
            pl.BlockSpec((1, 64), const),
            pl.BlockSpec((576, 64), const),
            pl.BlockSpec((1, 64), const),
            pl.BlockSpec((64, 256), const),
            pl.BlockSpec((1, 256), const),
            pl.BlockSpec((64, 256), const),
            pl.BlockSpec((1, 256), const),
            pl.BlockSpec((256, 128), const),
            pl.BlockSpec((1, 128), const),
        ],
        out_specs=(
            pl.BlockSpec((1, 29, 33, 128), img),
            pl.BlockSpec((1, 29, 33, 128), img),
            pl.BlockSpec((1, 29, 33, 128), img),
            pl.BlockSpec((1, 29, 33, 128), img),
            pl.BlockSpec((1, 28, 32, 256), img),
        ),
        scratch_shapes=[pltpu.VMEM((1, 58, 58, 64), jnp.bfloat16)],
        compiler_params=pltpu.CompilerParams(
            dimension_semantics=("parallel",), vmem_limit_bytes=_VMEM),
    )(x, c1[0].reshape(64, 64), c1[1].astype(jnp.float32).reshape(1, 64),
      c2[0].reshape(576, 64), c2[1].astype(jnp.float32).reshape(1, 64),
      c3[0].reshape(64, 256), c3[1].astype(jnp.float32).reshape(1, 256),
      down[0].reshape(64, 256), down[1].astype(jnp.float32).reshape(1, 256),
      nxt[0].reshape(256, 128), nxt[1].astype(jnp.float32).reshape(1, 128))


def _bs2_kernel(p00_ref, p01_ref, p10_ref, p11_ref, xd_ref,
                w2_ref, b2_ref, w3_ref, b3_ref, wd_ref, bd_ref, *rest,
                G, HO, WT, C, P, CO, PN, NXT):
    if PN:
        wn_ref, bn_ref = rest[0], rest[1]
        q00_ref, q01_ref, q10_ref, q11_ref, nxd_ref = rest[2:]
    else:
        (o_ref,) = rest
    M = G * HO * WT
    phases = (p00_ref, p01_ref, p10_ref, p11_ref)
    acc = None
    for dy in range(3):
        for dx in range(3):
            ph = phases[(dy % 2) * 2 + (dx % 2)]
            i0, j0 = dy // 2, dx // 2
            t = ph[:, i0:i0 + HO, j0:j0 + WT, :]
            wt = w2_ref[(dy * 3 + dx) * P:(dy * 3 + dx + 1) * P, :]
            d = jnp.dot(t.reshape(M, P), wt,
                        preferred_element_type=jnp.float32)
            acc = d if acc is None else acc + d
    y2 = jnp.maximum(acc + b2_ref[...], 0.0).astype(jnp.bfloat16)
    ident = jnp.dot(xd_ref[...].reshape(M, C), wd_ref[...],
                    preferred_element_type=jnp.float32) + bd_ref[...]
    out = jnp.dot(y2, w3_ref[...],
                  preferred_element_type=jnp.float32) + b3_ref[...]
    obf = jnp.maximum(out + ident, 0.0).astype(jnp.bfloat16)
    if PN:
        hp_n, wtp1_n, wt_n = NXT
        y1n = jnp.dot(obf, wn_ref[...], preferred_element_type=jnp.float32)
        y1n = jnp.maximum(y1n + bn_ref[...], 0.0).astype(jnp.bfloat16)
        yv = y1n.reshape(G, HO, WT, PN)[:, :, :HO, :]
        p00, p01, p10, p11 = _emit_phases(yv, hp_n, wtp1_n)
        q00_ref[...], q01_ref[...] = p00, p01
        q10_ref[...], q11_ref[...] = p10, p11
        ov = obf.reshape(G, HO, WT, CO)[:, :, :HO, :]
        nxd_ref[...] = _emit_xd(ov, wt_n)
    else:
        o_ref[...] = obf.reshape(G, HO, WT, CO)[:, :, :HO, :]


def _block_s2(pin, c2, c3, down, nxt, *, G, HO, WT, P, C, CO, nxt_geom):
    N = pin[0].shape[0]
    hp, wtp1 = pin[0].shape[1], pin[0].shape[2]
    PN = nxt[0].shape[3] if nxt is not None else 0
    const = lambda n: (0, 0)
    img = lambda n: (n, 0, 0, 0)
    inputs = list(pin) + [
        c2[0].reshape(9 * P, P), c2[1].astype(jnp.float32).reshape(1, P),
        c3[0].reshape(P, CO), c3[1].astype(jnp.float32).reshape(1, CO),
        down[0].reshape(C, CO), down[1].astype(jnp.float32).reshape(1, CO),
    ]
    in_specs = [pl.BlockSpec((G, hp, wtp1, P), img) for _ in range(4)] + [
        pl.BlockSpec((G, HO, WT, C), img),
        pl.BlockSpec((9 * P, P), const), pl.BlockSpec((1, P), const),
        pl.BlockSpec((P, CO), const), pl.BlockSpec((1, CO), const),
        pl.BlockSpec((C, CO), const), pl.BlockSpec((1, CO), const),
    ]
    if PN:
        hp_n, wtp1_n, wt_n = nxt_geom
        inputs += [nxt[0].reshape(CO, PN),
                   nxt[1].astype(jnp.float32).reshape(1, PN)]
        in_specs += [pl.BlockSpec((CO, PN), const),
                     pl.BlockSpec((1, PN), const)]
        out_shape = tuple(
            [jax.ShapeDtypeStruct((N, hp_n, wtp1_n, PN), jnp.bfloat16)] * 4
            + [jax.ShapeDtypeStruct((N, HO // 2, wt_n, CO), jnp.bfloat16)])
        out_specs = tuple(
            [pl.BlockSpec((G, hp_n, wtp1_n, PN), img)] * 4
            + [pl.BlockSpec((G, HO // 2, wt_n, CO), img)])
    else:
        out_shape = jax.ShapeDtypeStruct((N, HO, HO, CO), jnp.bfloat16)
        out_specs = pl.BlockSpec((G, HO, HO, CO), img)
    return pl.pallas_call(
        functools.partial(_bs2_kernel, G=G, HO=HO, WT=WT, C=C, P=P, CO=CO,
                          PN=PN, NXT=nxt_geom),
        out_shape=out_shape,
        grid=(N // G,),
        in_specs=in_specs,
        out_specs=out_specs,
        compiler_params=pltpu.CompilerParams(
            dimension_semantics=("parallel",), vmem_limit_bytes=_VMEM),
    )(*inputs)


def _pool_fc_kernel(x_ref, w_ref, b_ref, o_ref, *, hw):
    feats = jnp.sum(x_ref[...].astype(jnp.float32), axis=(1, 2)) * (1.0 / hw)
    o_ref[...] = jnp.dot(feats, w_ref[...],
                         preferred_element_type=jnp.float32) + b_ref[...]


def _avgpool_fc(x, fc_w, fc_b):
    N, H, W, C = x.shape
    ncls = fc_w.shape[1]
    Np = (ncls + 127) // 128 * 128
    w_p = jnp.pad(fc_w.astype(jnp.float32), ((0, 0), (0, Np - ncls)))
    b_p = jnp.pad(fc_b.astype(jnp.float32), (0, Np - ncls)).reshape(1, Np)
    out = pl.pallas_call(
        functools.partial(_pool_fc_kernel, hw=float(H * W)),
        out_shape=jax.ShapeDtypeStruct((N, Np), jnp.float32),
        grid=(1,),
        in_specs=[
            pl.BlockSpec((N, H, W, C), lambda i: (0, 0, 0, 0)),
            pl.BlockSpec((C, Np), lambda i: (0, 0)),
            pl.BlockSpec((1, Np), lambda i: (0, 0)),
        ],
        out_specs=pl.BlockSpec((N, Np), lambda i: (0, 0)),
        compiler_params=pltpu.CompilerParams(vmem_limit_bytes=_VMEM),
    )(x, w_p, b_p)
    return out[:, :ncls]


def kernel(x, conv1_w, conv1_b,
           b0_c1_w, b0_c1_b, b0_c2_w, b0_c2_b, b0_c3_w, b0_c3_b,
           b0_down_w, b0_down_b,
           b1_c1_w, b1_c1_b, b1_c2_w, b1_c2_b, b1_c3_w, b1_c3_b,
           b1_down_w, b1_down_b,
           b2_c1_w, b2_c1_b, b2_c2_w, b2_c2_b, b2_c3_w, b2_c3_b,
           b2_down_w, b2_down_b,
           b3_c1_w, b3_c1_b, b3_c2_w, b3_c2_b, b3_c3_w, b3_c3_b,
           b3_down_w, b3_down_b,
           fc_w, fc_b):
    h = _stem(x, conv1_w, conv1_b)
    t = _block0(h, (b0_c1_w, b0_c1_b), (b0_c2_w, b0_c2_b),
                (b0_c3_w, b0_c3_b), (b0_down_w, b0_down_b),
                (b1_c1_w, b1_c1_b))
    t = _block_s2(t, (b1_c2_w, b1_c2_b), (b1_c3_w, b1_c3_b),
                  (b1_down_w, b1_down_b), (b2_c1_w, b2_c1_b),
                  G=1, HO=28, WT=32, P=128, C=256, CO=512,
                  nxt_geom=(15, 17, 16))
    t = _block_s2(t, (b2_c2_w, b2_c2_b), (b2_c3_w, b2_c3_b),
                  (b2_down_w, b2_down_b), (b3_c1_w, b3_c1_b),
                  G=2, HO=14, WT=16, P=256, C=512, CO=1024,
                  nxt_geom=(8, 9, 8))
    h = _block_s2(t, (b3_c2_w, b3_c2_b), (b3_c3_w, b3_c3_b),
                  (b3_down_w, b3_down_b), None,
                  G=8, HO=7, WT=8, P=512, C=1024, CO=2048,
                  nxt_geom=None)
    return _avgpool_fc(h, fc_w, fc_b)


# speedup vs baseline: 24.5054x; 1.2085x over previous
import functools

import jax
import jax.numpy as jnp
from jax.experimental import pallas as pl
from jax.experimental.pallas import tpu as pltpu


_VMEM = 100 * 1024 * 1024


def _stem_kernel(x_ref, w_ref, b_ref, o_ref, s_ref):
    x = x_ref[0]
    accs = [None, None]
    for a in range(4):
        for b in range(4):
            wt = w_ref[(a * 4 + b) * 12:(a * 4 + b + 1) * 12, :]
            for pj in range(2):
                cc = b + pj
                p0, hh = cc // 2, cc % 2
                tap = x[a:a + 112, p0:p0 + 56, hh * 12:(hh + 1) * 12]
                d = jnp.dot(tap.reshape(6272, 12), wt,
                            preferred_element_type=jnp.float32)
                accs[pj] = d if accs[pj] is None else accs[pj] + d
    bias = b_ref[...]
    ye = jnp.maximum(accs[0] + bias, 0.0)
    yo = jnp.maximum(accs[1] + bias, 0.0)
    s_ref[...] = jnp.zeros_like(s_ref)
    ye5 = ye.reshape(56, 2, 56, 64)
    yo5 = yo.reshape(56, 2, 56, 64)
    s_ref[1:57, 0, 1:57, 0:64] = ye5[:, 0]
    s_ref[1:57, 1, 1:57, 0:64] = ye5[:, 1]
    s_ref[1:57, 0, 1:57, 64:128] = yo5[:, 0]
    s_ref[1:57, 1, 1:57, 64:128] = yo5[:, 1]
    m = None
    row_idx = ((0, 1), (1, 0), (1, 1))
    col_idx = ((0, 64), (1, 0), (1, 64))
    for a0, pa in row_idx:
        for p0, l0 in col_idx:
            t = s_ref[a0:a0 + 56, pa, p0:p0 + 56, l0:l0 + 64]
            m = t if m is None else jnp.maximum(m, t)
    o_ref[...] = m[None].astype(jnp.bfloat16)


def _stem(x_nchw, conv1_w, conv1_b):
    N = x_nchw.shape[0]
    xp = jnp.pad(x_nchw.astype(jnp.bfloat16),
                 ((0, 0), (0, 0), (3, 3), (3, 5)))
    xs = xp.reshape(N, 3, 115, 2, 58, 2, 2).transpose(0, 2, 4, 5, 3, 6, 1)
    xs = xs.reshape(N, 115, 58, 24)
    w = jnp.pad(conv1_w.astype(jnp.float32), ((0, 1), (0, 1), (0, 0), (0, 0)))
    w4 = w.reshape(4, 2, 4, 2, 3, 64).transpose(0, 2, 1, 3, 4, 5)
    w4 = w4.reshape(16 * 12, 64).astype(jnp.bfloat16)
    b2 = conv1_b.astype(jnp.float32).reshape(1, 64)
    return pl.pallas_call(
        _stem_kernel,
        out_shape=jax.ShapeDtypeStruct((N, 56, 56, 64), jnp.bfloat16),
        grid=(N,),
        in_specs=[
            pl.BlockSpec((1, 115, 58, 24), lambda n: (n, 0, 0, 0)),
            pl.BlockSpec((192, 64), lambda n: (0, 0)),
            pl.BlockSpec((1, 64), lambda n: (0, 0)),
        ],
        out_specs=pl.BlockSpec((1, 56, 56, 64), lambda n: (n, 0, 0, 0)),
        scratch_shapes=[pltpu.VMEM((58, 2, 58, 128), jnp.float32)],
        compiler_params=pltpu.CompilerParams(
            dimension_semantics=("parallel",), vmem_limit_bytes=_VMEM),
    )(xs, w4, b2)


def _emit_phases(y, hp, WTp1):
    G, Hn, _, Pn = y.shape
    h2 = Hn // 2
    re = y.reshape(G, h2, 2, Hn, Pn)
    evg = re[:, :, 0].reshape(G, h2, h2, 2 * Pn)
    odg = re[:, :, 1].reshape(G, h2, h2, 2 * Pn)
    rpad = hp - h2
    cpad = WTp1 - h2
    p11 = jnp.pad(evg[..., :Pn], ((0, 0), (0, rpad), (0, cpad), (0, 0)))
    p10 = jnp.pad(evg[..., Pn:], ((0, 0), (0, rpad), (1, cpad - 1), (0, 0)))
    p01 = jnp.pad(odg[..., :Pn], ((0, 0), (rpad, 0), (0, cpad), (0, 0)))
    p00 = jnp.pad(odg[..., Pn:], ((0, 0), (rpad, 0), (1, cpad - 1), (0, 0)))
    return p00, p01, p10, p11


def _emit_xd(o, WT):
    G, Hn, _, CO = o.shape
    h2 = Hn // 2
    ev = o.reshape(G, h2, 2, Hn, CO)[:, :, 0].reshape(G, h2, h2, 2 * CO)
    return jnp.pad(ev[..., :CO], ((0, 0), (0, 0), (0, WT - h2), (0, 0)))


def _b0_kernel(x_ref, w1_ref, b1_ref, w2_ref, b2_ref, w3_ref, b3_ref,
               wd_ref, bd_ref, wn_ref, bn_ref,
               p00_ref, p01_ref, p10_ref, p11_ref, xd_ref, y1p_ref):
    H, C, P, CO = 56, 64, 64, 256
    M = H * H
    x = x_ref[...].reshape(M, C)
    y1 = jnp.dot(x, w1_ref[...], preferred_element_type=jnp.float32)
    y1 = jnp.maximum(y1 + b1_ref[...], 0.0).astype(jnp.bfloat16)
    y1p_ref[...] = jnp.zeros_like(y1p_ref)
    y1p_ref[0, 1:H + 1, 1:H + 1, :] = y1.reshape(H, H, P)
    acc = None
    for dy in range(3):
        for dx in range(3):
            t = y1p_ref[0, dy:dy + H, dx:dx + H, :]
            wt = w2_ref[(dy * 3 + dx) * P:(dy * 3 + dx + 1) * P, :]
            d = jnp.dot(t.reshape(M, P), wt,
                        preferred_element_type=jnp.float32)
            acc = d if acc is None else acc + d
    y2 = jnp.maximum(acc + b2_ref[...], 0.0).astype(jnp.bfloat16)
    ident = jnp.dot(x, wd_ref[...],
                    preferred_element_type=jnp.float32) + bd_ref[...]
    out = jnp.dot(y2, w3_ref[...],
                  preferred_element_type=jnp.float32) + b3_ref[...]
    obf = jnp.maximum(out + ident, 0.0).astype(jnp.bfloat16)
    y1n = jnp.dot(obf, wn_ref[...], preferred_element_type=jnp.float32)
    y1n = jnp.maximum(y1n + bn_ref[...], 0.0).astype(jnp.bfloat16)
    p00, p01, p10, p11 = _emit_phases(y1n.reshape(1, H, H, 128), 29, 33)
    p00_ref[...], p01_ref[...] = p00, p01
    p10_ref[...], p11_ref[...] = p10, p11
    xd_ref[...] = _emit_xd(obf.reshape(1, H, H, CO), 32)


def _block0(x, c1, c2, c3, down, nxt):
    N = x.shape[0]
    const = lambda n: (0, 0)
    img = lambda n: (n, 0, 0, 0)
    return pl.pallas_call(
        _b0_kernel,
        out_shape=(
            jax.ShapeDtypeStruct((N, 29, 33, 128), jnp.bfloat16),
            jax.ShapeDtypeStruct((N, 29, 33, 128), jnp.bfloat16),
            jax.ShapeDtypeStruct((N, 29, 33, 128), jnp.bfloat16),
            jax.ShapeDtypeStruct((N, 29, 33, 128), jnp.bfloat16),
            jax.ShapeDtypeStruct((N, 28, 32, 256), jnp.bfloat16),
        ),
        grid=(N,),
        in_specs=[
            pl.BlockSpec((1, 56, 56, 64), img),
            pl.BlockSpec((64, 64), const),
            pl.BlockSpec((1, 64), const),
            pl.BlockSpec((576, 64), const),
            pl.BlockSpec((1, 64), const),
            pl.BlockSpec((64, 256), const),
            pl.BlockSpec((1, 256), const),
            pl.BlockSpec((64, 256), const),
            pl.BlockSpec((1, 256), const),
            pl.BlockSpec((256, 128), const),
            pl.BlockSpec((1, 128), const),
        ],
        out_specs=(
            pl.BlockSpec((1, 29, 33, 128), img),
            pl.BlockSpec((1, 29, 33, 128), img),
            pl.BlockSpec((1, 29, 33, 128), img),
            pl.BlockSpec((1, 29, 33, 128), img),
            pl.BlockSpec((1, 28, 32, 256), img),
        ),
        scratch_shapes=[pltpu.VMEM((1, 58, 58, 64), jnp.bfloat16)],
        compiler_params=pltpu.CompilerParams(
            dimension_semantics=("parallel",), vmem_limit_bytes=_VMEM),
    )(x, c1[0].reshape(64, 64), c1[1].astype(jnp.float32).reshape(1, 64),
      c2[0].reshape(576, 64), c2[1].astype(jnp.float32).reshape(1, 64),
      c3[0].reshape(64, 256), c3[1].astype(jnp.float32).reshape(1, 256),
      down[0].reshape(64, 256), down[1].astype(jnp.float32).reshape(1, 256),
      nxt[0].reshape(256, 128), nxt[1].astype(jnp.float32).reshape(1, 128))


def _bs2_kernel(p00_ref, p01_ref, p10_ref, p11_ref, xd_ref,
                w2_ref, b2_ref, w3_ref, b3_ref, wd_ref, bd_ref, *rest,
                G, HO, WT, C, P, CO, PN, NXT):
    if PN:
        wn_ref, bn_ref = rest[0], rest[1]
        q00_ref, q01_ref, q10_ref, q11_ref, nxd_ref = rest[2:]
    else:
        (o_ref,) = rest
    M = G * HO * WT
    phases = (p00_ref, p01_ref, p10_ref, p11_ref)
    acc = None
    for dy in range(3):
        for dx in range(3):
            ph = phases[(dy % 2) * 2 + (dx % 2)]
            i0, j0 = dy // 2, dx // 2
            t = ph[:, i0:i0 + HO, j0:j0 + WT, :]
            wt = w2_ref[(dy * 3 + dx) * P:(dy * 3 + dx + 1) * P, :]
            d = jnp.dot(t.reshape(M, P), wt,
                        preferred_element_type=jnp.float32)
            acc = d if acc is None else acc + d
    y2 = jnp.maximum(acc + b2_ref[...], 0.0).astype(jnp.bfloat16)
    ident = jnp.dot(xd_ref[...].reshape(M, C), wd_ref[...],
                    preferred_element_type=jnp.float32) + bd_ref[...]
    out = jnp.dot(y2, w3_ref[...],
                  preferred_element_type=jnp.float32) + b3_ref[...]
    obf = jnp.maximum(out + ident, 0.0).astype(jnp.bfloat16)
    if PN:
        hp_n, wtp1_n, wt_n = NXT
        y1n = jnp.dot(obf, wn_ref[...], preferred_element_type=jnp.float32)
        y1n = jnp.maximum(y1n + bn_ref[...], 0.0).astype(jnp.bfloat16)
        yv = y1n.reshape(G, HO, WT, PN)[:, :, :HO, :]
        p00, p01, p10, p11 = _emit_phases(yv, hp_n, wtp1_n)
        q00_ref[...], q01_ref[...] = p00, p01
        q10_ref[...], q11_ref[...] = p10, p11
        ov = obf.reshape(G, HO, WT, CO)[:, :, :HO, :]
        nxd_ref[...] = _emit_xd(ov, wt_n)
    else:
        o_ref[...] = obf.reshape(G, HO, WT, CO)[:, :, :HO, :]


def _block_s2(pin, c2, c3, down, nxt, *, G, HO, WT, P, C, CO, nxt_geom):
    N = pin[0].shape[0]
    hp, wtp1 = pin[0].shape[1], pin[0].shape[2]
    PN = nxt[0].shape[3] if nxt is not None else 0
    const = lambda n: (0, 0)
    img = lambda n: (n, 0, 0, 0)
    inputs = list(pin) + [
        c2[0].reshape(9 * P, P), c2[1].astype(jnp.float32).reshape(1, P),
        c3[0].reshape(P, CO), c3[1].astype(jnp.float32).reshape(1, CO),
        down[0].reshape(C, CO), down[1].astype(jnp.float32).reshape(1, CO),
    ]
    in_specs = [pl.BlockSpec((G, hp, wtp1, P), img) for _ in range(4)] + [
        pl.BlockSpec((G, HO, WT, C), img),
        pl.BlockSpec((9 * P, P), const), pl.BlockSpec((1, P), const),
        pl.BlockSpec((P, CO), const), pl.BlockSpec((1, CO), const),
        pl.BlockSpec((C, CO), const), pl.BlockSpec((1, CO), const),
    ]
    if PN:
        hp_n, wtp1_n, wt_n = nxt_geom
        inputs += [nxt[0].reshape(CO, PN),
                   nxt[1].astype(jnp.float32).reshape(1, PN)]
        in_specs += [pl.BlockSpec((CO, PN), const),
                     pl.BlockSpec((1, PN), const)]
        out_shape = tuple(
            [jax.ShapeDtypeStruct((N, hp_n, wtp1_n, PN), jnp.bfloat16)] * 4
            + [jax.ShapeDtypeStruct((N, HO // 2, wt_n, CO), jnp.bfloat16)])
        out_specs = tuple(
            [pl.BlockSpec((G, hp_n, wtp1_n, PN), img)] * 4
            + [pl.BlockSpec((G, HO // 2, wt_n, CO), img)])
    else:
        out_shape = jax.ShapeDtypeStruct((N, HO, HO, CO), jnp.bfloat16)
        out_specs = pl.BlockSpec((G, HO, HO, CO), img)
    return pl.pallas_call(
        functools.partial(_bs2_kernel, G=G, HO=HO, WT=WT, C=C, P=P, CO=CO,
                          PN=PN, NXT=nxt_geom),
        out_shape=out_shape,
        grid=(N // G,),
        in_specs=in_specs,
        out_specs=out_specs,
        compiler_params=pltpu.CompilerParams(
            dimension_semantics=("parallel",), vmem_limit_bytes=_VMEM),
    )(*inputs)


def _pool_fc_kernel(x_ref, w_ref, b_ref, o_ref, *, hw):
    feats = jnp.sum(x_ref[...].astype(jnp.float32), axis=(1, 2)) * (1.0 / hw)
    o_ref[...] = jnp.dot(feats, w_ref[...],
                         preferred_element_type=jnp.float32) + b_ref[...]


def _avgpool_fc(x, fc_w, fc_b):
    N, H, W, C = x.shape
    ncls = fc_w.shape[1]
    Np = (ncls + 127) // 128 * 128
    w_p = jnp.pad(fc_w.astype(jnp.float32), ((0, 0), (0, Np - ncls)))
    b_p = jnp.pad(fc_b.astype(jnp.float32), (0, Np - ncls)).reshape(1, Np)
    out = pl.pallas_call(
        functools.partial(_pool_fc_kernel, hw=float(H * W)),
        out_shape=jax.ShapeDtypeStruct((N, Np), jnp.float32),
        grid=(1,),
        in_specs=[
            pl.BlockSpec((N, H, W, C), lambda i: (0, 0, 0, 0)),
            pl.BlockSpec((C, Np), lambda i: (0, 0)),
            pl.BlockSpec((1, Np), lambda i: (0, 0)),
        ],
        out_specs=pl.BlockSpec((N, Np), lambda i: (0, 0)),
        compiler_params=pltpu.CompilerParams(vmem_limit_bytes=_VMEM),
    )(x, w_p, b_p)
    return out[:, :ncls]


def kernel(x, conv1_w, conv1_b,
           b0_c1_w, b0_c1_b, b0_c2_w, b0_c2_b, b0_c3_w, b0_c3_b,
           b0_down_w, b0_down_b,
           b1_c1_w, b1_c1_b, b1_c2_w, b1_c2_b, b1_c3_w, b1_c3_b,
           b1_down_w, b1_down_b,
           b2_c1_w, b2_c1_b, b2_c2_w, b2_c2_b, b2_c3_w, b2_c3_b,
           b2_down_w, b2_down_b,
           b3_c1_w, b3_c1_b, b3_c2_w, b3_c2_b, b3_c3_w, b3_c3_b,
           b3_down_w, b3_down_b,
           fc_w, fc_b):
    h = _stem(x, conv1_w, conv1_b)
    t = _block0(h, (b0_c1_w, b0_c1_b), (b0_c2_w, b0_c2_b),
                (b0_c3_w, b0_c3_b), (b0_down_w, b0_down_b),
                (b1_c1_w, b1_c1_b))
    t = _block_s2(t, (b1_c2_w, b1_c2_b), (b1_c3_w, b1_c3_b),
                  (b1_down_w, b1_down_b), (b2_c1_w, b2_c1_b),
                  G=1, HO=28, WT=32, P=128, C=256, CO=512,
                  nxt_geom=(15, 17, 16))
    t = _block_s2(t, (b2_c2_w, b2_c2_b), (b2_c3_w, b2_c3_b),
                  (b2_down_w, b2_down_b), (b3_c1_w, b3_c1_b),
                  G=2, HO=14, WT=16, P=256, C=512, CO=1024,
                  nxt_geom=(8, 9, 8))
    h = _block_s2(t, (b3_c2_w, b3_c2_b), (b3_c3_w, b3_c3_b),
                  (b3_down_w, b3_down_b), None,
                  G=8, HO=7, WT=8, P=512, C=1024, CO=2048,
                  nxt_geom=None)
    return _avgpool_fc(h, fc_w, fc_b)
